# Initial kernel scaffold; baseline (speedup 1.0000x reference)
import jax
import jax.numpy as jnp
from jax.experimental import pallas as pl


def kernel(x, topk_weights, topk_ids, w13, w2):
    raise NotImplementedError("write your pallas kernel here")


# speedup vs baseline: 2.2221x; 2.2221x over previous
import functools

import jax
import jax.numpy as jnp
from jax import lax
from jax.experimental import pallas as pl
from jax.experimental.pallas import tpu as pltpu
from jax.experimental.pallas import tpu_sc as plsc

T, H, I, E, K = 2048, 1024, 512, 16, 2
BLK = 128
NB = (T * K) // BLK + E
NPAD = NB * BLK
NW = 32


def _routing(topk_ids, topk_weights):
    ids = topk_ids.reshape(-1).astype(jnp.int32)
    counts = jnp.sum(ids[None, :] == jnp.arange(E, dtype=jnp.int32)[:, None],
                     axis=1).astype(jnp.int32)
    pcounts = ((counts + BLK - 1) // BLK) * BLK
    poffs = jnp.concatenate([jnp.zeros((1,), jnp.int32),
                             jnp.cumsum(pcounts)]).astype(jnp.int32)
    starts = jnp.concatenate([jnp.zeros((1,), jnp.int32),
                              jnp.cumsum(counts)]).astype(jnp.int32)
    order = jnp.argsort(ids, stable=True).astype(jnp.int32)
    sorted_ids = ids[order]
    rank = jnp.arange(T * K, dtype=jnp.int32) - starts[sorted_ids]
    dest_sorted = poffs[sorted_ids] + rank
    src_map = jnp.zeros((NPAD,), jnp.int32).at[dest_sorted].set(order // K)
    wsort = jnp.zeros((NPAD,), jnp.float32).at[dest_sorted].set(
        topk_weights.reshape(-1)[order])
    dest = jnp.zeros((T * K,), jnp.int32).at[order].set(dest_sorted)
    dest = dest.reshape(T, K)
    blk_expert = jnp.clip(
        jnp.searchsorted(poffs, jnp.arange(NB, dtype=jnp.int32) * BLK,
                         side="right").astype(jnp.int32) - 1, 0, E - 1)
    return src_map, wsort, dest[:, 0], dest[:, 1], blk_expert


def _sc_gather(x, src_map):
    mesh = plsc.VectorSubcoreMesh(core_axis_name="c", subcore_axis_name="s")
    rows_per_w = NPAD // NW
    CH = 64
    nch = rows_per_w // CH

    @functools.partial(
        pl.kernel, mesh=mesh,
        out_type=jax.ShapeDtypeStruct((NPAD, H), jnp.float32),
        scratch_types=[pltpu.VMEM((CH,), jnp.int32),
                       pltpu.VMEM((CH, H), jnp.float32),
                       pltpu.SemaphoreType.DMA],
    )
    def k(x_hbm, idx_hbm, out_hbm, idx_v, rows_v, sem):
        wid = lax.axis_index("s") * 2 + lax.axis_index("c")
        base = wid * rows_per_w

        def body(i, carry):
            off = base + i * CH
            pltpu.sync_copy(idx_hbm.at[pl.ds(off, CH)], idx_v)
            pltpu.async_copy(x_hbm.at[idx_v], rows_v, sem).wait()
            pltpu.sync_copy(rows_v, out_hbm.at[pl.ds(off, CH)])
            return carry

        lax.fori_loop(0, nch, body, 0)

    return k(x, src_map)


def _tc_moe(xs, w13, w2, wsort, blk_expert):
    def body(be_ref, xs_ref, w13_ref, w2_ref, ws_ref, out_ref):
        xsb = xs_ref[...]
        h = lax.dot_general(xsb, w13_ref[0], (((1,), (0,)), ((), ())),
                            preferred_element_type=jnp.float32)
        gate = h[:, :I]
        up = h[:, I:]
        act = gate * jax.nn.sigmoid(gate) * up
        o = lax.dot_general(act, w2_ref[0], (((1,), (0,)), ((), ())),
                            preferred_element_type=jnp.float32)
        out_ref[...] = o * ws_ref[...]

    grid_spec = pltpu.PrefetchScalarGridSpec(
        num_scalar_prefetch=1,
        grid=(NB,),
        in_specs=[
            pl.BlockSpec((BLK, H), lambda b, be: (b, 0)),
            pl.BlockSpec((1, H, 2 * I), lambda b, be: (be[b], 0, 0)),
            pl.BlockSpec((1, I, H), lambda b, be: (be[b], 0, 0)),
            pl.BlockSpec((BLK, 1), lambda b, be: (b, 0)),
        ],
        out_specs=pl.BlockSpec((BLK, H), lambda b, be: (b, 0)),
    )
    return pl.pallas_call(
        body, grid_spec=grid_spec,
        out_shape=jax.ShapeDtypeStruct((NPAD, H), jnp.float32),
    )(blk_expert, xs, w13, w2, wsort)


def _sc_combine(ys, dest0, dest1):
    mesh = plsc.VectorSubcoreMesh(core_axis_name="c", subcore_axis_name="s")
    tpw = T // NW
    CH = 32
    nch = tpw // CH

    @functools.partial(
        pl.kernel, mesh=mesh,
        out_type=jax.ShapeDtypeStruct((T, H), jnp.float32),
        scratch_types=[pltpu.VMEM((CH,), jnp.int32),
                       pltpu.VMEM((CH,), jnp.int32),
                       pltpu.VMEM((CH, H), jnp.float32),
                       pltpu.VMEM((CH, H), jnp.float32),
                       pltpu.SemaphoreType.DMA,
                       pltpu.SemaphoreType.DMA],
    )
    def k(ys_hbm, d0_hbm, d1_hbm, out_hbm, i0_v, i1_v, a_v, b_v, s0, s1):
        wid = lax.axis_index("s") * 2 + lax.axis_index("c")
        base = wid * tpw

        def body(ci, carry):
            off = base + ci * CH
            pltpu.sync_copy(d0_hbm.at[pl.ds(off, CH)], i0_v)
            pltpu.sync_copy(d1_hbm.at[pl.ds(off, CH)], i1_v)
            c0 = pltpu.async_copy(ys_hbm.at[i0_v], a_v, s0)
            c1 = pltpu.async_copy(ys_hbm.at[i1_v], b_v, s1)
            c0.wait()
            c1.wait()

            def row(r, rc):
                def col(c, cc):
                    sl = pl.ds(c * 16, 16)
                    a_v[r, sl] = a_v[r, sl] + b_v[r, sl]
                    return cc
                return lax.fori_loop(0, H // 16, col, rc)

            lax.fori_loop(0, CH, row, 0)
            pltpu.sync_copy(a_v, out_hbm.at[pl.ds(off, CH)])
            return carry

        lax.fori_loop(0, nch, body, 0)

    return k(ys, dest0, dest1)


def kernel(x, topk_weights, topk_ids, w13, w2):
    src_map, wsort, dest0, dest1, blk_expert = _routing(topk_ids, topk_weights)
    xs = _sc_gather(x, src_map)
    ys = _tc_moe(xs, w13, w2, wsort.reshape(NPAD, 1), blk_expert)
    return _sc_combine(ys, dest0, dest1)


# speedup vs baseline: 4.2299x; 1.9035x over previous
import functools

import jax
import jax.numpy as jnp
from jax import lax
from jax.experimental import pallas as pl
from jax.experimental.pallas import tpu as pltpu
from jax.experimental.pallas import tpu_sc as plsc

T, H, I, E, K = 2048, 1024, 512, 16, 2
BLK = 128
NB = (T * K) // BLK + E
NPAD = NB * BLK
NW = 32
TPW = T // NW


def _routing(topk_ids):
    ids = topk_ids.reshape(-1).astype(jnp.int32)
    oh = (ids[:, None] == jnp.arange(E, dtype=jnp.int32)[None, :]).astype(
        jnp.int32)
    incl = jnp.cumsum(oh, axis=0)
    counts = incl[-1]
    pcounts = ((counts + BLK - 1) // BLK) * BLK
    poffs = jnp.concatenate([jnp.zeros((1,), jnp.int32),
                             jnp.cumsum(pcounts)]).astype(jnp.int32)
    rank = jnp.sum(incl * oh, axis=1) - 1
    dest = jnp.sum(poffs[:E][None, :] * oh, axis=1) + rank
    blk_expert = jnp.clip(
        jnp.searchsorted(poffs, jnp.arange(NB, dtype=jnp.int32) * BLK,
                         side="right").astype(jnp.int32) - 1, 0, E - 1)
    return dest, blk_expert


def _sc_dispatch(x, dest0, dest1, w0, w1):
    mesh = plsc.VectorSubcoreMesh(core_axis_name="c", subcore_axis_name="s")

    @functools.partial(
        pl.kernel, mesh=mesh,
        out_type=(jax.ShapeDtypeStruct((NPAD, H), jnp.float32),
                  jax.ShapeDtypeStruct((NPAD,), jnp.float32)),
        scratch_types=[pltpu.VMEM((TPW,), jnp.int32),
                       pltpu.VMEM((TPW,), jnp.int32),
                       pltpu.VMEM((TPW,), jnp.float32),
                       pltpu.VMEM((TPW,), jnp.float32),
                       pltpu.VMEM((TPW, H), jnp.float32),
                       pltpu.SemaphoreType.DMA,
                       pltpu.SemaphoreType.DMA,
                       pltpu.SemaphoreType.DMA,
                       pltpu.SemaphoreType.DMA],
    )
    def k(x_hbm, d0_hbm, d1_hbm, w0_hbm, w1_hbm, xs_hbm, ws_hbm,
          i0_v, i1_v, w0_v, w1_v, rows_v, s0, s1, s2, s3):
        wid = lax.axis_index("s") * 2 + lax.axis_index("c")
        base = wid * TPW
        pltpu.sync_copy(d0_hbm.at[pl.ds(base, TPW)], i0_v)
        pltpu.sync_copy(d1_hbm.at[pl.ds(base, TPW)], i1_v)
        pltpu.sync_copy(w0_hbm.at[pl.ds(base, TPW)], w0_v)
        pltpu.sync_copy(w1_hbm.at[pl.ds(base, TPW)], w1_v)
        pltpu.sync_copy(x_hbm.at[pl.ds(base, TPW)], rows_v)
        c0 = pltpu.async_copy(rows_v, xs_hbm.at[i0_v], s0)
        c1 = pltpu.async_copy(rows_v, xs_hbm.at[i1_v], s1)
        c2 = pltpu.async_copy(w0_v, ws_hbm.at[i0_v], s2)
        c3 = pltpu.async_copy(w1_v, ws_hbm.at[i1_v], s3)
        c0.wait()
        c1.wait()
        c2.wait()
        c3.wait()

    return k(x, dest0, dest1, w0, w1)


def _tc_moe(xs, w13, w2, wsort, blk_expert):
    def body(be_ref, xs_ref, w13_ref, w2_ref, ws_ref, out_ref):
        xsb = xs_ref[...]
        h = lax.dot_general(xsb, w13_ref[0], (((1,), (0,)), ((), ())),
                            preferred_element_type=jnp.float32)
        gate = h[:, :I]
        up = h[:, I:]
        act = gate * jax.nn.sigmoid(gate) * up
        o = lax.dot_general(act, w2_ref[0], (((1,), (0,)), ((), ())),
                            preferred_element_type=jnp.float32)
        out_ref[...] = o * ws_ref[...]

    grid_spec = pltpu.PrefetchScalarGridSpec(
        num_scalar_prefetch=1,
        grid=(NB,),
        in_specs=[
            pl.BlockSpec((BLK, H), lambda b, be: (b, 0)),
            pl.BlockSpec((1, H, 2 * I), lambda b, be: (be[b], 0, 0)),
            pl.BlockSpec((1, I, H), lambda b, be: (be[b], 0, 0)),
            pl.BlockSpec((BLK, 1), lambda b, be: (b, 0)),
        ],
        out_specs=pl.BlockSpec((BLK, H), lambda b, be: (b, 0)),
    )
    return pl.pallas_call(
        body, grid_spec=grid_spec,
        out_shape=jax.ShapeDtypeStruct((NPAD, H), jnp.float32),
    )(blk_expert, xs, w13, w2, wsort)


def _sc_combine(ys, dest):
    mesh = plsc.VectorSubcoreMesh(core_axis_name="c", subcore_axis_name="s")
    CH = 32
    nch = TPW // CH

    @functools.partial(
        pl.kernel, mesh=mesh,
        out_type=jax.ShapeDtypeStruct((T, H), jnp.float32),
        scratch_types=[pltpu.VMEM((2 * CH,), jnp.int32),
                       pltpu.VMEM((2 * CH, H), jnp.float32),
                       pltpu.VMEM((CH, H), jnp.float32),
                       pltpu.SemaphoreType.DMA],
    )
    def k(ys_hbm, d_hbm, out_hbm, idx_v, pair_v, out_v, sem):
        wid = lax.axis_index("s") * 2 + lax.axis_index("c")
        base = wid * TPW

        def body(ci, carry):
            off = base + ci * CH
            pltpu.sync_copy(d_hbm.at[pl.ds(2 * off, 2 * CH)], idx_v)
            pltpu.async_copy(ys_hbm.at[idx_v], pair_v, sem).wait()

            def row(r, rc):
                @plsc.parallel_loop(0, H // 16, unroll=8)
                def col(c):
                    sl = pl.ds(c * 16, 16)
                    out_v[r, sl] = pair_v[2 * r, sl] + pair_v[2 * r + 1, sl]
                return rc

            lax.fori_loop(0, CH, row, 0)
            pltpu.sync_copy(out_v, out_hbm.at[pl.ds(off, CH)])
            return carry

        lax.fori_loop(0, nch, body, 0)

    return k(ys, dest)


def kernel(x, topk_weights, topk_ids, w13, w2):
    dest, blk_expert = _routing(topk_ids)
    dest2 = dest.reshape(T, K)
    w = topk_weights.astype(jnp.float32)
    xs, wsort = _sc_dispatch(x, dest2[:, 0], dest2[:, 1], w[:, 0], w[:, 1])
    ys = _tc_moe(xs, w13, w2, wsort.reshape(NPAD, 1), blk_expert)
    return _sc_combine(ys, dest)
